# baseline (speedup 1.0000x reference)
import jax
import jax.numpy as jnp
from jax import lax
from jax.experimental import pallas as pl
from jax.experimental.pallas import tpu as pltpu
from jax.experimental.pallas import tpu_sc as plsc

B, S = 4, 4096
WS, WH = 64, 128
V, TH = 1000000, 128

NC, NS = 2, 16
NW = NC * NS
WPB = NW // B
BPW = S // WPB
CH = 128
NCH = BPW // CH
WROWS = WS // WPB
SPC = [0, 2]


def _body(ids_hbm, ws_hbm, table_hbm, ws_out, emb_out, idx_v, rows_v, ws_v,
          rows_sh, *sems):
    gsems = sems[:NCH]
    ssems = {c: sems[NCH + i] for i, c in enumerate(SPC)}
    osem = sems[NCH + len(SPC)]
    wid = lax.axis_index("s") * NC + lax.axis_index("c")
    sub = lax.axis_index("s")
    batch = lax.div(wid, WPB)
    stripe = lax.rem(wid, WPB)

    pltpu.sync_copy(ids_hbm.at[batch, stripe], idx_v)

    gathers = [
        pltpu.async_copy(
            table_hbm.at[idx_v.at[j]],
            rows_v.at[pl.ds(j * CH, CH)],
            gsems[j],
        )
        for j in range(NCH)
    ]

    tok0 = stripe * BPW
    sp_copies = {}
    out_copies = []
    for j in range(NCH):
        gathers[j].wait()
        if j in SPC:
            slot = SPC.index(j)
            sp_copies[j] = pltpu.async_copy(
                rows_v.at[pl.ds(j * CH, CH)],
                rows_sh.at[sub, pl.ds(slot * CH, CH)],
                ssems[j],
            )
        else:
            out_copies.append(
                pltpu.async_copy(
                    rows_v.at[pl.ds(j * CH, CH)],
                    emb_out.at[batch, pl.ds(tok0 + j * CH, CH)],
                    osem,
                )
            )
    for j in SPC:
        slot = SPC.index(j)
        sp_copies[j].wait()
        out_copies.append(
            pltpu.async_copy(
                rows_sh.at[sub, pl.ds(slot * CH, CH)],
                emb_out.at[batch, pl.ds(tok0 + j * CH, CH)],
                osem,
            )
        )

    row0 = stripe * WROWS
    pltpu.sync_copy(ws_hbm.at[pl.ds(row0, WROWS)], ws_v)
    pltpu.sync_copy(ws_v, ws_out.at[batch, pl.ds(row0, WROWS)])

    for c in out_copies:
        c.wait()


@jax.jit
def _run(ids, ws, table):
    kern = pl.kernel(
        _body,
        out_type=(
            jax.ShapeDtypeStruct((B, WS, WH), jnp.float32),
            jax.ShapeDtypeStruct((B, S, TH), jnp.float32),
        ),
        mesh=plsc.VectorSubcoreMesh(core_axis_name="c", subcore_axis_name="s"),
        scratch_types=[
            pltpu.VMEM((NCH, CH), jnp.int32),
            pltpu.VMEM((BPW, TH), jnp.float32),
            pltpu.VMEM((WROWS, WH), jnp.float32),
            pltpu.VMEM_SHARED((NS, len(SPC) * CH, TH), jnp.float32),
        ] + [pltpu.SemaphoreType.DMA] * (NCH + len(SPC) + 1),
    )
    return kern(ids, ws, table)


def kernel(input_ids, attention_mask, init_workspace, word_table):
    ids = input_ids.reshape(B, WPB, NCH, CH)
    ws = init_workspace.reshape(WS, WH)
    workspace, embeddings = _run(ids, ws, word_table)
    return (workspace, embeddings)


# speedup vs baseline: 1.0347x; 1.0347x over previous
import jax
import jax.numpy as jnp
from jax import lax
from jax.experimental import pallas as pl
from jax.experimental.pallas import tpu as pltpu
from jax.experimental.pallas import tpu_sc as plsc

B, S = 4, 4096
WS, WH = 64, 128
V, TH = 1000000, 128

NC, NS = 2, 16
NW = NC * NS
WPB = NW // B
BPW = S // WPB
CH = 128
NCH = BPW // CH
WROWS = WS // WPB


def _body(ids_hbm, ws_hbm, table_hbm, ws_out, emb_out, idx_v, rows_v, ws_v,
          *sems):
    gsems, osem = sems[:NCH], sems[NCH]
    wid = lax.axis_index("s") * NC + lax.axis_index("c")
    batch = lax.div(wid, WPB)
    stripe = lax.rem(wid, WPB)

    pltpu.sync_copy(ids_hbm.at[batch, stripe], idx_v)

    gathers = []
    for j in range(NCH):
        gathers.append(
            pltpu.async_copy(
                table_hbm.at[idx_v.at[j]],
                rows_v.at[pl.ds(j * CH, CH)],
                gsems[j],
            )
        )

    tok0 = stripe * BPW
    out_copies = []
    for j in range(NCH):
        gathers[j].wait()
        out_copies.append(
            pltpu.async_copy(
                rows_v.at[pl.ds(j * CH, CH)],
                emb_out.at[batch, pl.ds(tok0 + j * CH, CH)],
                osem,
            )
        )

    row0 = stripe * WROWS
    pltpu.sync_copy(ws_hbm.at[pl.ds(row0, WROWS)], ws_v)
    pltpu.sync_copy(ws_v, ws_out.at[batch, pl.ds(row0, WROWS)])

    for c in out_copies:
        c.wait()


@jax.jit
def _run(ids, ws, table):
    kern = pl.kernel(
        _body,
        out_type=(
            jax.ShapeDtypeStruct((B, WS, WH), jnp.float32),
            jax.ShapeDtypeStruct((B, S, TH), jnp.float32),
        ),
        mesh=plsc.VectorSubcoreMesh(core_axis_name="c", subcore_axis_name="s"),
        scratch_types=[
            pltpu.VMEM((NCH, CH), jnp.int32),
            pltpu.VMEM((BPW, TH), jnp.float32),
            pltpu.VMEM((WROWS, WH), jnp.float32),
        ] + [pltpu.SemaphoreType.DMA] * (NCH + 1),
    )
    return kern(ids, ws, table)


def kernel(input_ids, attention_mask, init_workspace, word_table):
    ids = input_ids.reshape(B, WPB, NCH, CH)
    ws = init_workspace.reshape(WS, WH)
    workspace, embeddings = _run(ids, ws, word_table)
    return (workspace, embeddings)


# speedup vs baseline: 1.0377x; 1.0029x over previous
import jax
import jax.numpy as jnp
from jax import lax
from jax.experimental import pallas as pl
from jax.experimental.pallas import tpu as pltpu
from jax.experimental.pallas import tpu_sc as plsc

B, S = 4, 4096
WS, WH = 64, 128
V, TH = 1000000, 128

NC, NS = 2, 16
NW = NC * NS
WPB = NW // B
BPW = S // WPB
CH = 128
NCH = BPW // CH
WROWS = WS // WPB


def _body(ids_hbm, ws_hbm, table_hbm, ws_out, emb_out, idx_v, rows_v, ws_v,
          *sems):
    gsems, osem = sems[:NCH], sems[NCH]
    wid = lax.axis_index("c") * NS + lax.axis_index("s")
    batch = lax.div(wid, WPB)
    stripe = lax.rem(wid, WPB)

    pltpu.sync_copy(ids_hbm.at[batch, stripe], idx_v)

    gathers = []
    for j in range(NCH):
        gathers.append(
            pltpu.async_copy(
                table_hbm.at[idx_v.at[j]],
                rows_v.at[pl.ds(j * CH, CH)],
                gsems[j],
            )
        )

    tok0 = stripe * BPW
    out_copies = []
    for j in range(NCH):
        gathers[j].wait()
        out_copies.append(
            pltpu.async_copy(
                rows_v.at[pl.ds(j * CH, CH)],
                emb_out.at[batch, pl.ds(tok0 + j * CH, CH)],
                osem,
            )
        )

    row0 = stripe * WROWS
    pltpu.sync_copy(ws_hbm.at[pl.ds(row0, WROWS)], ws_v)
    pltpu.sync_copy(ws_v, ws_out.at[batch, pl.ds(row0, WROWS)])

    for c in out_copies:
        c.wait()


@jax.jit
def _run(ids, ws, table):
    kern = pl.kernel(
        _body,
        out_type=(
            jax.ShapeDtypeStruct((B, WS, WH), jnp.float32),
            jax.ShapeDtypeStruct((B, S, TH), jnp.float32),
        ),
        mesh=plsc.VectorSubcoreMesh(core_axis_name="c", subcore_axis_name="s"),
        scratch_types=[
            pltpu.VMEM((NCH, CH), jnp.int32),
            pltpu.VMEM((BPW, TH), jnp.float32),
            pltpu.VMEM((WROWS, WH), jnp.float32),
        ] + [pltpu.SemaphoreType.DMA] * (NCH + 1),
    )
    return kern(ids, ws, table)


def kernel(input_ids, attention_mask, init_workspace, word_table):
    ids = input_ids.reshape(B, WPB, NCH, CH)
    ws = init_workspace.reshape(WS, WH)
    workspace, embeddings = _run(ids, ws, word_table)
    return (workspace, embeddings)
